# baseline (speedup 1.0000x reference)
import jax
import jax.numpy as jnp
from jax.experimental import pallas as pl
from jax.experimental.pallas import tpu as pltpu

N = 100000
D_IN = 512
D_HID = 512
D_OUT = 768
BN = 4000
NB = N // BN


def _mlp_block(x_ref, w1_ref, b1_ref, w2_ref, b2_ref, o_ref, h_scr):
    i = pl.program_id(0)

    h = h_scr[(i + 1) % 2]
    o = jnp.dot(h, w2_ref[...], preferred_element_type=jnp.float32)
    t = jnp.tanh(o + b2_ref[...])
    o_ref[...] = 0.5 * t + 0.5

    x = x_ref[...].astype(jnp.bfloat16)
    h1 = jnp.dot(x, w1_ref[...],
                 preferred_element_type=jnp.float32).astype(jnp.bfloat16)
    h1 = h1 + b1_ref[...]
    h1 = jnp.where(h1 >= 0.0, h1, jnp.bfloat16(0.01) * h1)
    h_scr[i % 2] = h1


def kernel(encoded, W1, b1, W2, b2):
    w1b = W1.astype(jnp.bfloat16)
    b1b = b1.reshape(1, D_HID).astype(jnp.bfloat16)
    w2b = (0.5 * W2).astype(jnp.bfloat16)
    b2h = (0.5 * b2).reshape(1, D_OUT)
    grid = (NB + 1,)
    return pl.pallas_call(
        _mlp_block,
        grid=grid,
        in_specs=[
            pl.BlockSpec((BN, D_IN), lambda i: (jnp.minimum(i, NB - 1), 0)),
            pl.BlockSpec((D_IN, D_HID), lambda i: (0, 0)),
            pl.BlockSpec((1, D_HID), lambda i: (0, 0)),
            pl.BlockSpec((D_HID, D_OUT), lambda i: (0, 0)),
            pl.BlockSpec((1, D_OUT), lambda i: (0, 0)),
        ],
        out_specs=pl.BlockSpec((BN, D_OUT), lambda i: (jnp.maximum(i - 1, 0), 0)),
        out_shape=jax.ShapeDtypeStruct((N, D_OUT), jnp.float32),
        scratch_shapes=[pltpu.VMEM((2, BN, D_HID), jnp.bfloat16)],
        compiler_params=pltpu.CompilerParams(
            vmem_limit_bytes=100 * 1024 * 1024,
        ),
    )(encoded, w1b, b1b, w2b, b2h)


# speedup vs baseline: 1.0472x; 1.0472x over previous
import jax
import jax.numpy as jnp
from jax.experimental import pallas as pl
from jax.experimental.pallas import tpu as pltpu

N = 100000
D_IN = 512
D_HID = 512
D_OUT = 768
BN = 5000


def _mlp_block(x_ref, w1_ref, b1_ref, w2_ref, b2_ref, o_ref):
    x = x_ref[...].astype(jnp.bfloat16)
    h = jnp.dot(x, w1_ref[...],
                preferred_element_type=jnp.float32).astype(jnp.bfloat16)
    h = h + b1_ref[...]
    h = jnp.where(h >= 0.0, h, jnp.bfloat16(0.01) * h)
    o = jnp.dot(h, w2_ref[...], preferred_element_type=jnp.float32)
    t = jnp.tanh(o + b2_ref[...])
    o_ref[...] = 0.5 * t + 0.5


def kernel(encoded, W1, b1, W2, b2):
    w1b = W1.astype(jnp.bfloat16)
    b1b = b1.reshape(1, D_HID).astype(jnp.bfloat16)
    w2b = (0.5 * W2).astype(jnp.bfloat16)
    b2h = (0.5 * b2).reshape(1, D_OUT)
    grid = (N // BN,)
    return pl.pallas_call(
        _mlp_block,
        grid=grid,
        in_specs=[
            pl.BlockSpec((BN, D_IN), lambda i: (i, 0)),
            pl.BlockSpec((D_IN, D_HID), lambda i: (0, 0)),
            pl.BlockSpec((1, D_HID), lambda i: (0, 0)),
            pl.BlockSpec((D_HID, D_OUT), lambda i: (0, 0)),
            pl.BlockSpec((1, D_OUT), lambda i: (0, 0)),
        ],
        out_specs=pl.BlockSpec((BN, D_OUT), lambda i: (i, 0)),
        out_shape=jax.ShapeDtypeStruct((N, D_OUT), jnp.float32),
        compiler_params=pltpu.CompilerParams(
            vmem_limit_bytes=100 * 1024 * 1024,
        ),
    )(encoded, w1b, b1b, w2b, b2h)


# speedup vs baseline: 1.0520x; 1.0046x over previous
import jax
import jax.numpy as jnp
from jax.experimental import pallas as pl
from jax.experimental.pallas import tpu as pltpu

N = 100000
D_IN = 512
D_HID = 512
D_OUT = 768
BN = 4000


def _mlp_block(x_ref, w1_ref, b1_ref, w2_ref, b2_ref, o_ref, w1s, w2s):
    i = pl.program_id(0)

    @pl.when(i == 0)
    def _prep_weights():
        w1s[...] = w1_ref[...].astype(jnp.bfloat16)
        w2s[...] = (w2_ref[...] * 0.5).astype(jnp.bfloat16)

    b1b = b1_ref[...].astype(jnp.bfloat16)
    b2h = b2_ref[...] * 0.5

    x = x_ref[...].astype(jnp.bfloat16)
    h = jnp.dot(x, w1s[...],
                preferred_element_type=jnp.float32).astype(jnp.bfloat16)
    h = h + b1b
    h = jnp.where(h >= 0.0, h, jnp.bfloat16(0.01) * h)
    o = jnp.dot(h, w2s[...], preferred_element_type=jnp.float32)
    t = jnp.tanh(o + b2h)
    o_ref[...] = 0.5 * t + 0.5


def kernel(encoded, W1, b1, W2, b2):
    b1r = b1.reshape(1, D_HID)
    b2r = b2.reshape(1, D_OUT)
    grid = (N // BN,)
    return pl.pallas_call(
        _mlp_block,
        grid=grid,
        in_specs=[
            pl.BlockSpec((BN, D_IN), lambda i: (i, 0)),
            pl.BlockSpec((D_IN, D_HID), lambda i: (0, 0)),
            pl.BlockSpec((1, D_HID), lambda i: (0, 0)),
            pl.BlockSpec((D_HID, D_OUT), lambda i: (0, 0)),
            pl.BlockSpec((1, D_OUT), lambda i: (0, 0)),
        ],
        out_specs=pl.BlockSpec((BN, D_OUT), lambda i: (i, 0)),
        out_shape=jax.ShapeDtypeStruct((N, D_OUT), jnp.float32),
        scratch_shapes=[
            pltpu.VMEM((D_IN, D_HID), jnp.bfloat16),
            pltpu.VMEM((D_HID, D_OUT), jnp.bfloat16),
        ],
        compiler_params=pltpu.CompilerParams(
            vmem_limit_bytes=100 * 1024 * 1024,
        ),
    )(encoded, W1, b1r, W2, b2r)
